# baseline (speedup 1.0000x reference)
import functools

import jax
import jax.numpy as jnp
from jax import lax
from jax.experimental import pallas as pl
from jax.experimental.pallas import tpu as pltpu
from jax.experimental.pallas import tpu_sc as plsc

VOCAB = 100000
EMB = 64
HID = 512
NGRAM = 8
BATCH = 64
LOOKUPS = BATCH * NGRAM

VTILE = 2048
NFULL = VOCAB // VTILE
TAIL = VOCAB - NFULL * VTILE
NBUF = 4
NOBUF = 2


@functools.lru_cache(maxsize=None)
def _make_gather():
    info = plsc.get_sparse_core_info()
    nw = info.num_cores * info.num_subcores
    per_w = LOOKUPS // nw
    mesh = plsc.VectorSubcoreMesh(core_axis_name="c", subcore_axis_name="s")

    @functools.partial(
        pl.kernel,
        mesh=mesh,
        out_type=jax.ShapeDtypeStruct((LOOKUPS, EMB), jnp.float32),
        scratch_types=[
            pltpu.VMEM((per_w,), jnp.int32),
            pltpu.VMEM((per_w, EMB), jnp.float32),
            pltpu.SemaphoreType.DMA,
        ],
        compiler_params=pltpu.CompilerParams(use_tc_tiling_on_sc=False),
    )
    def gather(table_hbm, idx_hbm, out_hbm, idx_v, rows_v, sem):
        wid = lax.axis_index("s") * info.num_cores + lax.axis_index("c")
        base = wid * per_w
        pltpu.sync_copy(idx_hbm.at[pl.ds(base, per_w)], idx_v)
        pltpu.async_copy(table_hbm.at[idx_v], rows_v, sem).wait()
        pltpu.sync_copy(rows_v, out_hbm.at[pl.ds(base, per_w)])

    return gather


def _mlp_body(h0_ref, W1_ref, b1_ref, b2m_ref, b2t_ref, w2_hbm, out_hbm,
              h_scr, w2buf, obuf, tailbuf, otailbuf, insem, outsem, tailsem):
    for k in range(NBUF):
        pltpu.make_async_copy(
            w2_hbm.at[:, pl.ds(k * VTILE, VTILE)], w2buf.at[k], insem.at[k]
        ).start()

    h = jnp.dot(h0_ref[...], W1_ref[...],
                preferred_element_type=jnp.float32) + b1_ref[...]
    h_scr[...] = jnp.maximum(h, 0.0)

    def step(t, _):
        slot = lax.rem(t, NBUF)
        oslot = lax.rem(t, NOBUF)
        pltpu.make_async_copy(
            w2_hbm.at[:, pl.ds(t * VTILE, VTILE)], w2buf.at[slot],
            insem.at[slot]).wait()

        @pl.when(t >= NOBUF)
        def _():
            pltpu.make_async_copy(
                obuf.at[oslot],
                out_hbm.at[:, pl.ds((t - NOBUF) * VTILE, VTILE)],
                outsem.at[oslot]).wait()

        obuf[oslot] = (
            jnp.dot(h_scr[...], w2buf[slot],
                    preferred_element_type=jnp.float32)
            + b2m_ref[t]
        )
        pltpu.make_async_copy(
            obuf.at[oslot], out_hbm.at[:, pl.ds(t * VTILE, VTILE)],
            outsem.at[oslot]).start()

        @pl.when(t + NBUF < NFULL)
        def _():
            pltpu.make_async_copy(
                w2_hbm.at[:, pl.ds((t + NBUF) * VTILE, VTILE)],
                w2buf.at[slot], insem.at[slot]).start()

        @pl.when(t + NBUF == NFULL)
        def _():
            pltpu.make_async_copy(
                w2_hbm.at[:, pl.ds(NFULL * VTILE, TAIL)],
                tailbuf, tailsem).start()

        return 0

    lax.fori_loop(0, NFULL, step, 0)

    pltpu.make_async_copy(
        w2_hbm.at[:, pl.ds(NFULL * VTILE, TAIL)], tailbuf, tailsem).wait()
    for t in (NFULL - 2, NFULL - 1):
        pltpu.make_async_copy(
            obuf.at[t % NOBUF], out_hbm.at[:, pl.ds(t * VTILE, VTILE)],
            outsem.at[t % NOBUF]).wait()
    otailbuf[...] = (
        jnp.dot(h_scr[...], tailbuf[...],
                preferred_element_type=jnp.float32)
        + b2t_ref[...]
    )
    cp = pltpu.make_async_copy(
        otailbuf, out_hbm.at[:, pl.ds(NFULL * VTILE, TAIL)], outsem.at[0])
    cp.start()
    cp.wait()


def kernel(x, emb, W1, b1, W2, b2):
    idx = x.reshape(-1).astype(jnp.int32)
    rows = _make_gather()(emb, idx)
    h0 = rows.reshape(BATCH, NGRAM * EMB)

    out = pl.pallas_call(
        _mlp_body,
        in_specs=[
            pl.BlockSpec(memory_space=pltpu.MemorySpace.VMEM),
            pl.BlockSpec(memory_space=pltpu.MemorySpace.VMEM),
            pl.BlockSpec(memory_space=pltpu.MemorySpace.VMEM),
            pl.BlockSpec(memory_space=pltpu.MemorySpace.VMEM),
            pl.BlockSpec(memory_space=pltpu.MemorySpace.VMEM),
            pl.BlockSpec(memory_space=pltpu.MemorySpace.HBM),
        ],
        out_specs=pl.BlockSpec(memory_space=pltpu.MemorySpace.HBM),
        out_shape=jax.ShapeDtypeStruct((BATCH, VOCAB), jnp.float32),
        scratch_shapes=[
            pltpu.VMEM((BATCH, HID), jnp.float32),
            pltpu.VMEM((NBUF, HID, VTILE), jnp.float32),
            pltpu.VMEM((NOBUF, BATCH, VTILE), jnp.float32),
            pltpu.VMEM((HID, TAIL), jnp.float32),
            pltpu.VMEM((BATCH, TAIL), jnp.float32),
            pltpu.SemaphoreType.DMA((NBUF,)),
            pltpu.SemaphoreType.DMA((NOBUF,)),
            pltpu.SemaphoreType.DMA,
        ],
        compiler_params=pltpu.CompilerParams(
            vmem_limit_bytes=100 * 1024 * 1024),
    )(h0, W1, b1.reshape(1, HID),
      b2[:NFULL * VTILE].reshape(NFULL, 1, VTILE),
      b2[NFULL * VTILE:].reshape(1, TAIL), W2)
    return out
